# baseline (speedup 1.0000x reference)
import dataclasses
import functools

import jax
import jax.numpy as jnp
from jax import lax
from jax.experimental import pallas as pl
from jax.experimental.pallas import tpu as pltpu
from jax.experimental.pallas import tpu_sc as plsc

OUT_K = 256
IN_K = 1024
NLANES = 16

_MESH = plsc.VectorSubcoreMesh(core_axis_name="c", subcore_axis_name="s")
_CP = pltpu.CompilerParams()
if "needs_layout_passes" in pltpu.CompilerParams.__dataclass_fields__:
    _CP = dataclasses.replace(_CP, needs_layout_passes=False)


def _norm_key_body(w0, w1, w2, w3, w4, w5, w6, w7, out):
    s0 = w0[...] * w0[...]
    s1 = w1[...] * w1[...]
    s2 = w2[...] * w2[...]
    s3 = w3[...] * w3[...]
    s4 = w4[...] * w4[...]
    s5 = w5[...] * w5[...]
    s6 = w6[...] * w6[...]
    s7 = w7[...] * w7[...]
    acc = ((s0 + s4) + (s2 + s6)) + ((s1 + s5) + (s3 + s7))
    norm = jnp.sqrt(acc)
    out[...] = jnp.bitwise_not(lax.bitcast_convert_type(norm, jnp.int32))


def _norm_keys(wc, rows):
    block = 256
    grid = rows // block
    spec = pl.BlockSpec((block, IN_K), lambda i: (i, 0))
    return pl.pallas_call(
        _norm_key_body,
        grid=(grid,),
        in_specs=[spec] * 8,
        out_specs=spec,
        out_shape=jax.ShapeDtypeStruct((rows, IN_K), jnp.int32),
    )(*wc)


def _sc_topk_gather(keys, posf, wtsf, rows):
    rows_per = rows // 32

    @functools.partial(
        pl.kernel,
        out_type=(
            jax.ShapeDtypeStruct((rows * OUT_K * 3,), jnp.float32),
            jax.ShapeDtypeStruct((rows * OUT_K * 8,), jnp.float32),
        ),
        mesh=_MESH,
        compiler_params=_CP,
        scratch_types=[
            pltpu.VMEM((IN_K,), jnp.int32),
            pltpu.VMEM((IN_K,), jnp.int32),
            pltpu.VMEM((IN_K,), jnp.int32),
            pltpu.VMEM((IN_K,), jnp.int32),
            pltpu.VMEM((256,), jnp.int32),
            pltpu.VMEM((256,), jnp.int32),
            pltpu.VMEM((IN_K * 8,), jnp.float32),
            pltpu.VMEM((IN_K * 3,), jnp.float32),
            pltpu.VMEM((OUT_K * 8,), jnp.float32),
            pltpu.VMEM((OUT_K * 3,), jnp.float32),
            pltpu.SemaphoreType.DMA,
            pltpu.SemaphoreType.DMA,
            pltpu.SemaphoreType.DMA,
            pltpu.SemaphoreType.DMA,
        ],
    )
    def k(keys_hbm, pos_hbm, wts_hbm, outp_hbm, outw_hbm,
          key_a, idx_a, key_b, idx_b, hist, offs, wrow, prow, wout, pout,
          sem_k, sem_w, sem_p, sem_o):
        wid = lax.axis_index("c") * 16 + lax.axis_index("s")

        @pl.loop(0, rows_per)
        def _row(r):
            row = wid * rows_per + r
            pltpu.async_copy(keys_hbm.at[row], key_a, sem_k).wait()
            cw = pltpu.async_copy(wts_hbm.at[pl.ds(row * (IN_K * 8), IN_K * 8)],
                                  wrow, sem_w)
            cp = pltpu.async_copy(pos_hbm.at[pl.ds(row * (IN_K * 3), IN_K * 3)],
                                  prow, sem_p)

            for p in range(4):
                src_k, src_i = (key_a, idx_a) if p % 2 == 0 else (key_b, idx_b)
                dst_k, dst_i = (key_b, idx_b) if p % 2 == 0 else (key_a, idx_a)
                shift = 8 * p

                for j in range(16):
                    hist[pl.ds(16 * j, 16)] = jnp.zeros((16,), jnp.int32)

                @pl.loop(0, IN_K, step=NLANES)
                def _hist(c0):
                    kk = src_k[pl.ds(c0, NLANES)]
                    d = lax.shift_right_logical(kk, shift) & 255
                    cnt, lastm = plsc.scan_count(d)
                    plsc.addupdate_scatter(hist, [d], cnt.astype(jnp.int32),
                                           mask=lastm)

                carry = jnp.int32(0)
                for j in range(16):
                    h = hist[pl.ds(16 * j, 16)]
                    offs[pl.ds(16 * j, 16)] = plsc.cumsum(h) - h + carry
                    carry = carry + jnp.sum(h)

                @pl.loop(0, IN_K, step=NLANES)
                def _perm(c0):
                    kk = src_k[pl.ds(c0, NLANES)]
                    if p == 0:
                        vv = lax.iota(jnp.int32, NLANES) + c0
                    else:
                        vv = src_i[pl.ds(c0, NLANES)]
                    d = lax.shift_right_logical(kk, shift) & 255
                    cnt, lastm = plsc.scan_count(d)
                    cnt = cnt.astype(jnp.int32)
                    base = plsc.load_gather(offs, [d])
                    pos = base + cnt - 1
                    plsc.store_scatter(dst_k, [pos], kk)
                    plsc.store_scatter(dst_i, [pos], vv)
                    plsc.addupdate_scatter(offs, [d], cnt, mask=lastm)

            cw.wait()
            cp.wait()

            for i in range(OUT_K // NLANES):
                sel = idx_a[pl.ds(NLANES * i, NLANES)]
                dst = lax.iota(jnp.int32, NLANES) + NLANES * i
                for c in range(8):
                    vals = plsc.load_gather(wrow, [sel * 8 + c])
                    plsc.store_scatter(wout, [dst * 8 + c], vals)
                for c in range(3):
                    vals = plsc.load_gather(prow, [sel * 3 + c])
                    plsc.store_scatter(pout, [dst * 3 + c], vals)

            pltpu.async_copy(
                pout, outp_hbm.at[pl.ds(row * (OUT_K * 3), OUT_K * 3)],
                sem_o).wait()
            pltpu.async_copy(
                wout, outw_hbm.at[pl.ds(row * (OUT_K * 8), OUT_K * 8)],
                sem_o).wait()

    return k(keys, posf, wtsf)


def kernel(positions, weights):
    b, c, in_k, _ = positions.shape
    rows = b * c
    wc = [weights[..., i].reshape(rows, in_k) for i in range(8)]
    keys = _norm_keys(wc, rows)
    posf = positions.reshape(rows * in_k * 3)
    wtsf = weights.reshape(rows * in_k * 8)
    outp, outw = _sc_topk_gather(keys, posf, wtsf, rows)
    return (outp.reshape(b, c, OUT_K, 3), outw.reshape(b, c, OUT_K, 8))
